# Initial kernel scaffold; baseline (speedup 1.0000x reference)
import jax
import jax.numpy as jnp
from jax.experimental import pallas as pl


def kernel(x, Wqkv_w, Wqkv_b, sel_w, out_w, out_b):
    raise NotImplementedError("write your pallas kernel here")


# speedup vs baseline: 2.9131x; 2.9131x over previous
import math
import functools

import jax
import jax.numpy as jnp
from jax.experimental import pallas as pl
from jax.experimental.pallas import tpu as pltpu

B, L, D, H = 2, 2048, 1024, 16
HD = D // H
NPAIR = H // 2
K_TOP = math.ceil(math.sqrt(L))
KSEL = 64
LBLK = 512
MBLK = 256
SCALE = 1.0 / math.sqrt(HD)

F32 = jnp.float32


def _kv_sel_kernel(x_ref, wk_ref, wv_ref, bk_ref, bv_ref, selw_ref,
                   k_ref, v_ref, sel_ref):
    x = x_ref[0]
    kk = jax.lax.dot_general(x, wk_ref[...], (((1,), (1,)), ((), ())),
                             preferred_element_type=F32) + bk_ref[...]
    vv = jax.lax.dot_general(x, wv_ref[...], (((1,), (1,)), ((), ())),
                             preferred_element_type=F32) + bv_ref[...]
    for g in range(NPAIR):
        k_ref[0, g] = kk[:, 128 * g:128 * (g + 1)]
        v_ref[0, g] = vv[:, 128 * g:128 * (g + 1)]
    s = jnp.sum(x * selw_ref[...], axis=1, keepdims=True)
    sel_ref[0] = s


def _kv_sel(x, wk, wv, bk, bv, selw):
    nl = L // LBLK
    return pl.pallas_call(
        _kv_sel_kernel,
        grid=(B, nl),
        in_specs=[
            pl.BlockSpec((1, LBLK, D), lambda b, i: (b, i, 0)),
            pl.BlockSpec((D, D), lambda b, i: (0, 0)),
            pl.BlockSpec((D, D), lambda b, i: (0, 0)),
            pl.BlockSpec((1, D), lambda b, i: (0, 0)),
            pl.BlockSpec((1, D), lambda b, i: (0, 0)),
            pl.BlockSpec((1, D), lambda b, i: (0, 0)),
        ],
        out_specs=[
            pl.BlockSpec((1, NPAIR, LBLK, 128), lambda b, i: (b, 0, i, 0)),
            pl.BlockSpec((1, NPAIR, LBLK, 128), lambda b, i: (b, 0, i, 0)),
            pl.BlockSpec((1, LBLK, 1), lambda b, i: (b, i, 0)),
        ],
        out_shape=[
            jax.ShapeDtypeStruct((B, NPAIR, L, 128), F32),
            jax.ShapeDtypeStruct((B, NPAIR, L, 128), F32),
            jax.ShapeDtypeStruct((B, L, 1), F32),
        ],
    )(x, wk, wv, bk, bv, selw)


def _topk_kernel(sel_ref, idx_ref):
    vals0 = sel_ref[0]
    lane = jax.lax.broadcasted_iota(jnp.int32, (1, L), 1)
    lane64 = jax.lax.broadcasted_iota(jnp.int32, (1, KSEL), 1)

    def body(i, carry):
        vals, idxv = carry
        m = jnp.max(vals)
        cand = jnp.where(vals == m, lane, L)
        j = jnp.min(cand)
        idxv = jnp.where(lane64 == i, j, idxv)
        vals = jnp.where(lane == j, -1e30, vals)
        return vals, idxv

    idx0 = jnp.zeros((1, KSEL), jnp.int32)
    _, idxv = jax.lax.fori_loop(0, K_TOP, body, (vals0, idx0))
    idx_ref[0] = idxv


def _topk(sel_row):
    return pl.pallas_call(
        _topk_kernel,
        grid=(B,),
        in_specs=[pl.BlockSpec((1, 1, L), lambda b: (b, 0, 0))],
        out_specs=pl.BlockSpec((1, 1, KSEL), lambda b: (b, 0, 0)),
        out_shape=jax.ShapeDtypeStruct((B, 1, KSEL), jnp.int32),
    )(sel_row)


def _gather_kernel(idx_ref, x_ref, out_ref):
    out_ref[...] = x_ref[...]


def _gather_rows(x_rows, idx_flat):
    grid_spec = pltpu.PrefetchScalarGridSpec(
        num_scalar_prefetch=1,
        grid=(B * KSEL,),
        in_specs=[
            pl.BlockSpec((1, 1, D),
                         lambda g, idx: (g // KSEL * L + idx[g], 0, 0)),
        ],
        out_specs=pl.BlockSpec((1, 1, D), lambda g, idx: (g, 0, 0)),
    )
    return pl.pallas_call(
        _gather_kernel,
        grid_spec=grid_spec,
        out_shape=jax.ShapeDtypeStruct((B * KSEL, 1, D), F32),
    )(idx_flat, x_rows)


def _attn_kernel(xs_ref, wq_ref, bq_ref, k_ref, v_ref, t_ref, ctx_ref):
    xs = xs_ref[0]
    t = t_ref[0]
    col = jax.lax.broadcasted_iota(jnp.int32, (KSEL, L), 1)
    causal = jnp.where(col <= t, 0.0, -10000.0)
    lane128 = jax.lax.broadcasted_iota(jnp.int32, (1, 128), 1)
    m0 = (lane128 < HD).astype(F32)
    m1 = 1.0 - m0
    for g in range(NPAIR):
        q = jax.lax.dot_general(xs, wq_ref[g], (((1,), (1,)), ((), ())),
                                preferred_element_type=F32) + bq_ref[g]
        kp = k_ref[0, g]
        vp = v_ref[0, g]
        ctx_pair = jnp.zeros((KSEL, 128), F32)
        for mask in (m0, m1):
            s = jax.lax.dot_general(q * mask, kp, (((1,), (1,)), ((), ())),
                                    preferred_element_type=F32)
            s = s * SCALE + causal
            s = s - jnp.max(s, axis=1, keepdims=True)
            p = jnp.exp(s)
            p = p / jnp.sum(p, axis=1, keepdims=True)
            ctx_pair = ctx_pair + jax.lax.dot_general(
                p, vp * mask, (((1,), (0,)), ((), ())),
                preferred_element_type=F32)
        ctx_ref[0, g] = ctx_pair


def _attention(x_sel, wq3, bq3, k, v, t_col):
    return pl.pallas_call(
        _attn_kernel,
        grid=(B,),
        in_specs=[
            pl.BlockSpec((1, KSEL, D), lambda b: (b, 0, 0)),
            pl.BlockSpec((NPAIR, 128, D), lambda b: (0, 0, 0)),
            pl.BlockSpec((NPAIR, 1, 128), lambda b: (0, 0, 0)),
            pl.BlockSpec((1, NPAIR, L, 128), lambda b: (b, 0, 0, 0)),
            pl.BlockSpec((1, NPAIR, L, 128), lambda b: (b, 0, 0, 0)),
            pl.BlockSpec((1, KSEL, 1), lambda b: (b, 0, 0)),
        ],
        out_specs=pl.BlockSpec((1, NPAIR, KSEL, 128), lambda b: (b, 0, 0, 0)),
        out_shape=jax.ShapeDtypeStruct((B, NPAIR, KSEL, 128), F32),
    )(x_sel, wq3, bq3, k, v, t_col)


def _outproj_kernel(ctx_ref, ow_ref, ob_ref, xs_ref, selw_ref, src_ref):
    ctx = jnp.concatenate([ctx_ref[0, g] for g in range(NPAIR)], axis=1)
    attn = jax.lax.dot_general(ctx, ow_ref[...], (((1,), (1,)), ((), ())),
                               preferred_element_type=F32) + ob_ref[...]
    logit = jnp.sum(xs_ref[0] * selw_ref[...], axis=1, keepdims=True)
    src_ref[0] = attn * jax.nn.sigmoid(logit)


def _outproj(ctx, out_w, out_b, x_sel, selw):
    return pl.pallas_call(
        _outproj_kernel,
        grid=(B,),
        in_specs=[
            pl.BlockSpec((1, NPAIR, KSEL, 128), lambda b: (b, 0, 0, 0)),
            pl.BlockSpec((D, D), lambda b: (0, 0)),
            pl.BlockSpec((1, D), lambda b: (0, 0)),
            pl.BlockSpec((1, KSEL, D), lambda b: (b, 0, 0)),
            pl.BlockSpec((1, D), lambda b: (0, 0)),
        ],
        out_specs=pl.BlockSpec((1, KSEL, D), lambda b: (b, 0, 0)),
        out_shape=jax.ShapeDtypeStruct((B, KSEL, D), F32),
    )(ctx, out_w, out_b, x_sel, selw)


def _merge_kernel(x_ref, src_ref, idx_ref, y_ref):
    base = pl.program_id(1) * MBLK
    rows = jax.lax.broadcasted_iota(jnp.int32, (MBLK, KSEL), 0) + base
    cols = jax.lax.broadcasted_iota(jnp.int32, (MBLK, KSEL), 1)
    idxr = idx_ref[0]
    p = jnp.logical_and(rows == idxr, cols < K_TOP).astype(F32)
    y_ref[0] = x_ref[0] + jax.lax.dot_general(
        p, src_ref[0], (((1,), (0,)), ((), ())), preferred_element_type=F32)


def _merge(x, src, idx):
    nm = L // MBLK
    return pl.pallas_call(
        _merge_kernel,
        grid=(B, nm),
        in_specs=[
            pl.BlockSpec((1, MBLK, D), lambda b, i: (b, i, 0)),
            pl.BlockSpec((1, KSEL, D), lambda b, i: (b, 0, 0)),
            pl.BlockSpec((1, 1, KSEL), lambda b, i: (b, 0, 0)),
        ],
        out_specs=pl.BlockSpec((1, MBLK, D), lambda b, i: (b, i, 0)),
        out_shape=jax.ShapeDtypeStruct((B, L, D), F32),
    )(x, src, idx)


@jax.jit
def kernel(x, Wqkv_w, Wqkv_b, sel_w, out_w, out_b):
    wq = Wqkv_w[:D]
    wk = Wqkv_w[D:2 * D]
    wv = Wqkv_w[2 * D:]
    bq = Wqkv_b[:D].reshape(NPAIR, 1, 128)
    bk = Wqkv_b[D:2 * D].reshape(1, D)
    bv = Wqkv_b[2 * D:].reshape(1, D)
    selw = sel_w.reshape(1, D)
    obr = out_b.reshape(1, D)
    wq3 = wq.reshape(NPAIR, 128, D)

    k, v, sel = _kv_sel(x, wk, wv, bk, bv, selw)
    sel_row = sel.reshape(B, 1, L)
    idx = _topk(sel_row)
    idx_flat = idx.reshape(B * KSEL)
    x_rows = x.reshape(B * L, 1, D)
    x_sel = _gather_rows(x_rows, idx_flat).reshape(B, KSEL, D)
    t_col = idx.reshape(B, KSEL, 1)
    ctx = _attention(x_sel, wq3, bq, k, v, t_col)
    src = _outproj(ctx, out_w, obr, x_sel, selw)
    return _merge(x, src, idx)


# speedup vs baseline: 6.0635x; 2.0815x over previous
import math
import functools

import jax
import jax.numpy as jnp
from jax.experimental import pallas as pl
from jax.experimental.pallas import tpu as pltpu

B, L, D, H = 2, 2048, 1024, 16
HD = D // H
NPAIR = H // 2
K_TOP = math.ceil(math.sqrt(L))
KSEL = 64
LBLK = 512
MBLK = 256
SCALE = 1.0 / math.sqrt(HD)

F32 = jnp.float32


def _kv_sel_kernel(x_ref, wk_ref, wv_ref, bk_ref, bv_ref, selw_ref,
                   k_ref, v_ref, sel_ref):
    x = x_ref[0]
    kk = jax.lax.dot_general(x, wk_ref[...], (((1,), (1,)), ((), ())),
                             preferred_element_type=F32) + bk_ref[...]
    vv = jax.lax.dot_general(x, wv_ref[...], (((1,), (1,)), ((), ())),
                             preferred_element_type=F32) + bv_ref[...]
    for g in range(NPAIR):
        k_ref[0, g] = kk[:, 128 * g:128 * (g + 1)]
        v_ref[0, g] = vv[:, 128 * g:128 * (g + 1)]
    s = jnp.sum(x * selw_ref[...], axis=1, keepdims=True)
    sel_ref[0] = s


def _kv_sel(x, wk, wv, bk, bv, selw):
    nl = L // LBLK
    return pl.pallas_call(
        _kv_sel_kernel,
        grid=(B, nl),
        in_specs=[
            pl.BlockSpec((1, LBLK, D), lambda b, i: (b, i, 0)),
            pl.BlockSpec((D, D), lambda b, i: (0, 0)),
            pl.BlockSpec((D, D), lambda b, i: (0, 0)),
            pl.BlockSpec((1, D), lambda b, i: (0, 0)),
            pl.BlockSpec((1, D), lambda b, i: (0, 0)),
            pl.BlockSpec((1, D), lambda b, i: (0, 0)),
        ],
        out_specs=[
            pl.BlockSpec((1, NPAIR, LBLK, 128), lambda b, i: (b, 0, i, 0)),
            pl.BlockSpec((1, NPAIR, LBLK, 128), lambda b, i: (b, 0, i, 0)),
            pl.BlockSpec((1, LBLK, 1), lambda b, i: (b, i, 0)),
        ],
        out_shape=[
            jax.ShapeDtypeStruct((B, NPAIR, L, 128), F32),
            jax.ShapeDtypeStruct((B, NPAIR, L, 128), F32),
            jax.ShapeDtypeStruct((B, L, 1), F32),
        ],
    )(x, wk, wv, bk, bv, selw)


SB = 8
SL = L // SB


def _topk_gather_kernel(sel_ref, x_ref, idx_ref, t_ref, xsel_ref):
    vals0 = sel_ref[...]
    r_iota = jax.lax.broadcasted_iota(jnp.int32, (B, SB, SL), 1)
    c_iota = jax.lax.broadcasted_iota(jnp.int32, (B, SB, SL), 2)
    pos = r_iota * SL + c_iota
    lane64 = jax.lax.broadcasted_iota(jnp.int32, (B, 1, KSEL), 2)

    def body(i, carry):
        vals, idxv = carry
        m = jnp.max(jnp.max(vals, axis=2, keepdims=True), axis=1,
                    keepdims=True)
        cand = jnp.where(vals == m, pos, L)
        jv = jnp.min(jnp.min(cand, axis=2, keepdims=True), axis=1,
                     keepdims=True)
        idxv = jnp.where(lane64 == i, jv, idxv)
        vals = jnp.where(pos == jv, -1e30, vals)
        return vals, idxv

    idx0 = jnp.zeros((B, 1, KSEL), jnp.int32)
    _, idxv = jax.lax.fori_loop(0, K_TOP, body, (vals0, idx0))
    idx_ref[...] = idxv

    eye_r = jax.lax.broadcasted_iota(jnp.int32, (KSEL, KSEL), 0)
    eye_c = jax.lax.broadcasted_iota(jnp.int32, (KSEL, KSEL), 1)
    eye = (eye_r == eye_c).astype(F32)
    idx_f = idxv.astype(F32)
    col_f = jax.lax.broadcasted_iota(jnp.int32, (KSEL, L), 1).astype(F32)
    for b in range(B):
        t_col = jax.lax.dot_general(eye, idx_f[b], (((1,), (1,)), ((), ())),
                                    preferred_element_type=F32)
        t_ref[b] = t_col
        p = (col_f == t_col).astype(F32)
        xsel_ref[b] = jax.lax.dot_general(
            p, x_ref[b], (((1,), (0,)), ((), ())), preferred_element_type=F32)


def _topk_gather(sel3, x):
    return pl.pallas_call(
        _topk_gather_kernel,
        grid=(1,),
        in_specs=[
            pl.BlockSpec((B, SB, SL), lambda i: (0, 0, 0)),
            pl.BlockSpec((B, L, D), lambda i: (0, 0, 0)),
        ],
        out_specs=[
            pl.BlockSpec((B, 1, KSEL), lambda i: (0, 0, 0)),
            pl.BlockSpec((B, KSEL, 1), lambda i: (0, 0, 0)),
            pl.BlockSpec((B, KSEL, D), lambda i: (0, 0, 0)),
        ],
        out_shape=[
            jax.ShapeDtypeStruct((B, 1, KSEL), jnp.int32),
            jax.ShapeDtypeStruct((B, KSEL, 1), F32),
            jax.ShapeDtypeStruct((B, KSEL, D), F32),
        ],
    )(sel3, x)


def _attn_kernel(xs_ref, wq_ref, bq_ref, k_ref, v_ref, t_ref, ctx_ref):
    xs = xs_ref[0]
    t = t_ref[0]
    col = jax.lax.broadcasted_iota(jnp.int32, (KSEL, L), 1).astype(F32)
    causal = jnp.where(col <= t, 0.0, -10000.0)
    lane128 = jax.lax.broadcasted_iota(jnp.int32, (1, 128), 1)
    m0 = (lane128 < HD).astype(F32)
    m1 = 1.0 - m0
    for g in range(NPAIR):
        q = jax.lax.dot_general(xs, wq_ref[g], (((1,), (1,)), ((), ())),
                                preferred_element_type=F32) + bq_ref[g]
        kp = k_ref[0, g]
        vp = v_ref[0, g]
        ctx_pair = jnp.zeros((KSEL, 128), F32)
        for mask in (m0, m1):
            s = jax.lax.dot_general(q * mask, kp, (((1,), (1,)), ((), ())),
                                    preferred_element_type=F32)
            s = s * SCALE + causal
            s = s - jnp.max(s, axis=1, keepdims=True)
            p = jnp.exp(s)
            p = p / jnp.sum(p, axis=1, keepdims=True)
            ctx_pair = ctx_pair + jax.lax.dot_general(
                p, vp * mask, (((1,), (0,)), ((), ())),
                preferred_element_type=F32)
        ctx_ref[0, g] = ctx_pair


def _attention(x_sel, wq3, bq3, k, v, t_col):
    return pl.pallas_call(
        _attn_kernel,
        grid=(B,),
        in_specs=[
            pl.BlockSpec((1, KSEL, D), lambda b: (b, 0, 0)),
            pl.BlockSpec((NPAIR, 128, D), lambda b: (0, 0, 0)),
            pl.BlockSpec((NPAIR, 1, 128), lambda b: (0, 0, 0)),
            pl.BlockSpec((1, NPAIR, L, 128), lambda b: (b, 0, 0, 0)),
            pl.BlockSpec((1, NPAIR, L, 128), lambda b: (b, 0, 0, 0)),
            pl.BlockSpec((1, KSEL, 1), lambda b: (b, 0, 0)),
        ],
        out_specs=pl.BlockSpec((1, NPAIR, KSEL, 128), lambda b: (b, 0, 0, 0)),
        out_shape=jax.ShapeDtypeStruct((B, NPAIR, KSEL, 128), F32),
    )(x_sel, wq3, bq3, k, v, t_col)


def _outproj_kernel(ctx_ref, ow_ref, ob_ref, xs_ref, selw_ref, src_ref):
    ctx = jnp.concatenate([ctx_ref[0, g] for g in range(NPAIR)], axis=1)
    attn = jax.lax.dot_general(ctx, ow_ref[...], (((1,), (1,)), ((), ())),
                               preferred_element_type=F32) + ob_ref[...]
    logit = jnp.sum(xs_ref[0] * selw_ref[...], axis=1, keepdims=True)
    src_ref[0] = attn * jax.nn.sigmoid(logit)


def _outproj(ctx, out_w, out_b, x_sel, selw):
    return pl.pallas_call(
        _outproj_kernel,
        grid=(B,),
        in_specs=[
            pl.BlockSpec((1, NPAIR, KSEL, 128), lambda b: (b, 0, 0, 0)),
            pl.BlockSpec((D, D), lambda b: (0, 0)),
            pl.BlockSpec((1, D), lambda b: (0, 0)),
            pl.BlockSpec((1, KSEL, D), lambda b: (b, 0, 0)),
            pl.BlockSpec((1, D), lambda b: (0, 0)),
        ],
        out_specs=pl.BlockSpec((1, KSEL, D), lambda b: (b, 0, 0)),
        out_shape=jax.ShapeDtypeStruct((B, KSEL, D), F32),
    )(ctx, out_w, out_b, x_sel, selw)


def _merge_kernel(x_ref, src_ref, idx_ref, y_ref):
    base = pl.program_id(1) * MBLK
    rows = jax.lax.broadcasted_iota(jnp.int32, (MBLK, KSEL), 0) + base
    cols = jax.lax.broadcasted_iota(jnp.int32, (MBLK, KSEL), 1)
    idxr = idx_ref[0]
    p = jnp.logical_and(rows == idxr, cols < K_TOP).astype(F32)
    y_ref[0] = x_ref[0] + jax.lax.dot_general(
        p, src_ref[0], (((1,), (0,)), ((), ())), preferred_element_type=F32)


def _merge(x, src, idx):
    nm = L // MBLK
    return pl.pallas_call(
        _merge_kernel,
        grid=(B, nm),
        in_specs=[
            pl.BlockSpec((1, MBLK, D), lambda b, i: (b, i, 0)),
            pl.BlockSpec((1, KSEL, D), lambda b, i: (b, 0, 0)),
            pl.BlockSpec((1, 1, KSEL), lambda b, i: (b, 0, 0)),
        ],
        out_specs=pl.BlockSpec((1, MBLK, D), lambda b, i: (b, i, 0)),
        out_shape=jax.ShapeDtypeStruct((B, L, D), F32),
    )(x, src, idx)


@jax.jit
def kernel(x, Wqkv_w, Wqkv_b, sel_w, out_w, out_b):
    wq = Wqkv_w[:D]
    wk = Wqkv_w[D:2 * D]
    wv = Wqkv_w[2 * D:]
    bq = Wqkv_b[:D].reshape(NPAIR, 1, 128)
    bk = Wqkv_b[D:2 * D].reshape(1, D)
    bv = Wqkv_b[2 * D:].reshape(1, D)
    selw = sel_w.reshape(1, D)
    obr = out_b.reshape(1, D)
    wq3 = wq.reshape(NPAIR, 128, D)

    k, v, sel = _kv_sel(x, wk, wv, bk, bv, selw)
    sel3 = sel.reshape(B, SB, SL)
    idx, t_col, x_sel = _topk_gather(sel3, x)
    ctx = _attention(x_sel, wq3, bq, k, v, t_col)
    src = _outproj(ctx, out_w, obr, x_sel, selw)
    return _merge(x, src, idx)


# speedup vs baseline: 6.4988x; 1.0718x over previous
import math

import jax
import jax.numpy as jnp
from jax.experimental import pallas as pl
from jax.experimental.pallas import tpu as pltpu

B, L, D, H = 2, 2048, 1024, 16
HD = D // H
NPAIR = H // 2
K_TOP = math.ceil(math.sqrt(L))
KSEL = 64
LBLK = 512
MBLK = 256
SCALE = 1.0 / math.sqrt(HD)

F32 = jnp.float32
BF16 = jnp.bfloat16


def _kv_sel_kernel(x_ref, wk_ref, wv_ref, bk_ref, bv_ref, selw_ref,
                   k_ref, v_ref, sel_ref):
    x = x_ref[0]
    xb = x.astype(BF16)
    kk = jax.lax.dot_general(xb, wk_ref[...], (((1,), (1,)), ((), ())),
                             preferred_element_type=F32) + bk_ref[...]
    vv = jax.lax.dot_general(xb, wv_ref[...], (((1,), (1,)), ((), ())),
                             preferred_element_type=F32) + bv_ref[...]
    kkb = kk.astype(BF16)
    vvb = vv.astype(BF16)
    for g in range(NPAIR):
        k_ref[0, g] = kkb[:, 128 * g:128 * (g + 1)]
        v_ref[0, g] = vvb[:, 128 * g:128 * (g + 1)]
    s = jnp.sum(x * selw_ref[...], axis=1, keepdims=True)
    sel_ref[0] = s


def _kv_sel(x, wkb, wvb, bk, bv, selw):
    nl = L // LBLK
    return pl.pallas_call(
        _kv_sel_kernel,
        grid=(B, nl),
        in_specs=[
            pl.BlockSpec((1, LBLK, D), lambda b, i: (b, i, 0)),
            pl.BlockSpec((D, D), lambda b, i: (0, 0)),
            pl.BlockSpec((D, D), lambda b, i: (0, 0)),
            pl.BlockSpec((1, D), lambda b, i: (0, 0)),
            pl.BlockSpec((1, D), lambda b, i: (0, 0)),
            pl.BlockSpec((1, D), lambda b, i: (0, 0)),
        ],
        out_specs=[
            pl.BlockSpec((1, NPAIR, LBLK, 128), lambda b, i: (b, 0, i, 0)),
            pl.BlockSpec((1, NPAIR, LBLK, 128), lambda b, i: (b, 0, i, 0)),
            pl.BlockSpec((1, LBLK, 1), lambda b, i: (b, i, 0)),
        ],
        out_shape=[
            jax.ShapeDtypeStruct((B, NPAIR, L, 128), BF16),
            jax.ShapeDtypeStruct((B, NPAIR, L, 128), BF16),
            jax.ShapeDtypeStruct((B, L, 1), F32),
        ],
    )(x, wkb, wvb, bk, bv, selw)


SB = 8
SL = L // SB


def _topk_gather_kernel(sel_ref, x_ref, idx_ref, t_ref, xsel_ref):
    vals0 = sel_ref[...]
    r_iota = jax.lax.broadcasted_iota(jnp.int32, (B, SB, SL), 1)
    c_iota = jax.lax.broadcasted_iota(jnp.int32, (B, SB, SL), 2)
    pos = r_iota * SL + c_iota
    lane64 = jax.lax.broadcasted_iota(jnp.int32, (B, 1, KSEL), 2)

    def body(i, carry):
        vals, idxv = carry
        m = jnp.max(jnp.max(vals, axis=2, keepdims=True), axis=1,
                    keepdims=True)
        cand = jnp.where(vals == m, pos, L)
        jv = jnp.min(jnp.min(cand, axis=2, keepdims=True), axis=1,
                     keepdims=True)
        idxv = jnp.where(lane64 == i, jv, idxv)
        vals = jnp.where(pos == jv, -1e30, vals)
        return vals, idxv

    idx0 = jnp.zeros((B, 1, KSEL), jnp.int32)
    _, idxv = jax.lax.fori_loop(0, K_TOP, body, (vals0, idx0))
    idx_ref[...] = idxv

    eye_r = jax.lax.broadcasted_iota(jnp.int32, (KSEL, KSEL), 0)
    eye_c = jax.lax.broadcasted_iota(jnp.int32, (KSEL, KSEL), 1)
    eye = (eye_r == eye_c).astype(F32)
    idx_f = idxv.astype(F32)
    col_f = jax.lax.broadcasted_iota(jnp.int32, (KSEL, L), 1).astype(F32)
    for b in range(B):
        t_col = jax.lax.dot_general(eye, idx_f[b], (((1,), (1,)), ((), ())),
                                    preferred_element_type=F32)
        t_ref[b] = t_col
        p = (col_f == t_col).astype(F32)
        xsel_ref[b] = jax.lax.dot_general(
            p, x_ref[b], (((1,), (0,)), ((), ())), preferred_element_type=F32)


def _topk_gather(sel3, x):
    return pl.pallas_call(
        _topk_gather_kernel,
        grid=(1,),
        in_specs=[
            pl.BlockSpec((B, SB, SL), lambda i: (0, 0, 0)),
            pl.BlockSpec((B, L, D), lambda i: (0, 0, 0)),
        ],
        out_specs=[
            pl.BlockSpec((B, 1, KSEL), lambda i: (0, 0, 0)),
            pl.BlockSpec((B, KSEL, 1), lambda i: (0, 0, 0)),
            pl.BlockSpec((B, KSEL, D), lambda i: (0, 0, 0)),
        ],
        out_shape=[
            jax.ShapeDtypeStruct((B, 1, KSEL), jnp.int32),
            jax.ShapeDtypeStruct((B, KSEL, 1), F32),
            jax.ShapeDtypeStruct((B, KSEL, D), F32),
        ],
    )(sel3, x)


def _attn_kernel(xs_ref, wq_ref, bq_ref, k_ref, v_ref, t_ref, ctx_ref):
    xs = xs_ref[0].astype(BF16)
    t = t_ref[0]
    col = jax.lax.broadcasted_iota(jnp.int32, (KSEL, L), 1).astype(F32)
    causal = jnp.where(col <= t, 0.0, -10000.0)
    lane128 = jax.lax.broadcasted_iota(jnp.int32, (1, 128), 1)
    m0 = (lane128 < HD).astype(F32)
    m1 = 1.0 - m0
    m0b = m0.astype(BF16)
    m1b = m1.astype(BF16)
    for g in range(NPAIR):
        q = jax.lax.dot_general(xs, wq_ref[g], (((1,), (1,)), ((), ())),
                                preferred_element_type=F32) + bq_ref[g]
        kp = k_ref[0, g]
        vp = v_ref[0, g]
        ctx_pair = jnp.zeros((KSEL, 128), F32)
        for mask, maskb in ((m0, m0b), (m1, m1b)):
            qb = (q * mask).astype(BF16)
            s = jax.lax.dot_general(qb, kp, (((1,), (1,)), ((), ())),
                                    preferred_element_type=F32)
            s = s * SCALE + causal
            s = s - jnp.max(s, axis=1, keepdims=True)
            p = jnp.exp(s)
            p = p / jnp.sum(p, axis=1, keepdims=True)
            ctx_pair = ctx_pair + jax.lax.dot_general(
                p.astype(BF16), vp * maskb, (((1,), (0,)), ((), ())),
                preferred_element_type=F32)
        ctx_ref[0, g] = ctx_pair


def _attention(x_sel, wq3b, bq3, k, v, t_col):
    return pl.pallas_call(
        _attn_kernel,
        grid=(B,),
        in_specs=[
            pl.BlockSpec((1, KSEL, D), lambda b: (b, 0, 0)),
            pl.BlockSpec((NPAIR, 128, D), lambda b: (0, 0, 0)),
            pl.BlockSpec((NPAIR, 1, 128), lambda b: (0, 0, 0)),
            pl.BlockSpec((1, NPAIR, L, 128), lambda b: (b, 0, 0, 0)),
            pl.BlockSpec((1, NPAIR, L, 128), lambda b: (b, 0, 0, 0)),
            pl.BlockSpec((1, KSEL, 1), lambda b: (b, 0, 0)),
        ],
        out_specs=pl.BlockSpec((1, NPAIR, KSEL, 128), lambda b: (b, 0, 0, 0)),
        out_shape=jax.ShapeDtypeStruct((B, NPAIR, KSEL, 128), F32),
    )(x_sel, wq3b, bq3, k, v, t_col)


def _outproj_kernel(ctx_ref, ow_ref, ob_ref, xs_ref, selw_ref, src_ref):
    ctx = jnp.concatenate([ctx_ref[0, g] for g in range(NPAIR)], axis=1)
    attn = jax.lax.dot_general(ctx.astype(BF16), ow_ref[...],
                               (((1,), (1,)), ((), ())),
                               preferred_element_type=F32) + ob_ref[...]
    logit = jnp.sum(xs_ref[0] * selw_ref[...], axis=1, keepdims=True)
    src_ref[0] = attn * jax.nn.sigmoid(logit)


def _outproj(ctx, out_wb, out_b, x_sel, selw):
    return pl.pallas_call(
        _outproj_kernel,
        grid=(B,),
        in_specs=[
            pl.BlockSpec((1, NPAIR, KSEL, 128), lambda b: (b, 0, 0, 0)),
            pl.BlockSpec((D, D), lambda b: (0, 0)),
            pl.BlockSpec((1, D), lambda b: (0, 0)),
            pl.BlockSpec((1, KSEL, D), lambda b: (b, 0, 0)),
            pl.BlockSpec((1, D), lambda b: (0, 0)),
        ],
        out_specs=pl.BlockSpec((1, KSEL, D), lambda b: (b, 0, 0)),
        out_shape=jax.ShapeDtypeStruct((B, KSEL, D), F32),
    )(ctx, out_wb, out_b, x_sel, selw)


def _merge_kernel(x_ref, src_ref, idx_ref, y_ref):
    base = pl.program_id(1) * MBLK
    rows = jax.lax.broadcasted_iota(jnp.int32, (MBLK, KSEL), 0) + base
    cols = jax.lax.broadcasted_iota(jnp.int32, (MBLK, KSEL), 1)
    idxr = idx_ref[0]
    p = jnp.logical_and(rows == idxr, cols < K_TOP).astype(BF16)
    y_ref[0] = x_ref[0] + jax.lax.dot_general(
        p, src_ref[0].astype(BF16), (((1,), (0,)), ((), ())),
        preferred_element_type=F32)


def _merge(x, src, idx):
    nm = L // MBLK
    return pl.pallas_call(
        _merge_kernel,
        grid=(B, nm),
        in_specs=[
            pl.BlockSpec((1, MBLK, D), lambda b, i: (b, i, 0)),
            pl.BlockSpec((1, KSEL, D), lambda b, i: (b, 0, 0)),
            pl.BlockSpec((1, 1, KSEL), lambda b, i: (b, 0, 0)),
        ],
        out_specs=pl.BlockSpec((1, MBLK, D), lambda b, i: (b, i, 0)),
        out_shape=jax.ShapeDtypeStruct((B, L, D), F32),
    )(x, src, idx)


@jax.jit
def kernel(x, Wqkv_w, Wqkv_b, sel_w, out_w, out_b):
    wq = Wqkv_w[:D]
    wk = Wqkv_w[D:2 * D]
    wv = Wqkv_w[2 * D:]
    bq = Wqkv_b[:D].reshape(NPAIR, 1, 128)
    bk = Wqkv_b[D:2 * D].reshape(1, D)
    bv = Wqkv_b[2 * D:].reshape(1, D)
    selw = sel_w.reshape(1, D)
    obr = out_b.reshape(1, D)
    wq3b = wq.reshape(NPAIR, 128, D).astype(BF16)
    wkb = wk.astype(BF16)
    wvb = wv.astype(BF16)
    out_wb = out_w.astype(BF16)

    k, v, sel = _kv_sel(x, wkb, wvb, bk, bv, selw)
    sel3 = sel.reshape(B, SB, SL)
    idx, t_col, x_sel = _topk_gather(sel3, x)
    ctx = _attention(x_sel, wq3b, bq, k, v, t_col)
    src = _outproj(ctx, out_wb, obr, x_sel, selw)
    return _merge(x, src, idx)
